# baseline (speedup 1.0000x reference)
import functools

import jax
import jax.numpy as jnp
from jax import lax
from jax.experimental import pallas as pl
from jax.experimental.pallas import tpu as pltpu
from jax.experimental.pallas import tpu_sc as plsc

B, T = 1024, 4096
NW = 32
RPW = B // NW
LN = 16
NV = T // LN


def _wid():
    return lax.axis_index("s") * 2 + lax.axis_index("c")


def _stats_body(x_hbm, lensx_hbm, part_hbm, lxv, buf0, buf1, pbuf, s0, s1):
    base = _wid() * RPW
    pltpu.sync_copy(lensx_hbm.at[pl.ds(base * LN, RPW * LN)], lxv)
    bufs, sems = (buf0, buf1), (s0, s1)

    sv = jnp.zeros((LN,), jnp.float32)
    qv = jnp.zeros((LN,), jnp.float32)
    pltpu.async_copy(x_hbm.at[base], buf0, s0)
    for r in range(RPW):
        b = r % 2
        pltpu.make_async_copy(x_hbm.at[base + r], bufs[b], sems[b]).wait()
        if r + 1 < RPW:
            pltpu.async_copy(x_hbm.at[base + r + 1], bufs[(r + 1) % 2],
                             sems[(r + 1) % 2])
        buf = bufs[b]
        lenv = lxv[pl.ds(r * LN, LN)]
        col0 = lax.broadcasted_iota(jnp.int32, (LN,), 0)

        def inner(j, carry):
            col, s, q = carry
            x = buf[pl.ds(j * LN, LN)]
            xm = jnp.where(col < lenv, x, 0.0)
            return col + LN, s + xm, q + xm * x

        _, sv, qv = lax.fori_loop(0, NV, inner, (col0, sv, qv))

    pbuf[0, :] = sv
    pbuf[1, :] = qv
    pltpu.sync_copy(pbuf, part_hbm.at[pl.ds(2 * _wid(), 2)])


def _norm_body(x_hbm, lensx_hbm, ac_hbm, o_hbm,
               lxv, ac, buf0, buf1, si0, si1, so0, so1):
    base = _wid() * RPW
    pltpu.sync_copy(lensx_hbm.at[pl.ds(base * LN, RPW * LN)], lxv)
    pltpu.sync_copy(ac_hbm, ac)
    a_v = ac[0, :]
    c_v = ac[1, :]

    bufs, isems, osems = (buf0, buf1), (si0, si1), (so0, so1)
    pltpu.async_copy(x_hbm.at[base], buf0, si0)
    for r in range(RPW):
        b = r % 2
        pltpu.make_async_copy(x_hbm.at[base + r], bufs[b], isems[b]).wait()
        if r + 1 < RPW:
            nb = (r + 1) % 2
            if r >= 1:
                pltpu.make_async_copy(bufs[nb], o_hbm.at[pl.ds(0, T)],
                                      osems[nb]).wait()
            pltpu.async_copy(x_hbm.at[base + r + 1], bufs[nb], isems[nb])
        buf = bufs[b]
        lenv = lxv[pl.ds(r * LN, LN)]
        col0 = lax.broadcasted_iota(jnp.int32, (LN,), 0)

        def inner(j, col):
            x = buf[pl.ds(j * LN, LN)]
            y = x * a_v + c_v
            buf[pl.ds(j * LN, LN)] = jnp.where(col < lenv, y, x)
            return col + LN

        lax.fori_loop(0, NV, inner, col0)
        pltpu.async_copy(buf, o_hbm.at[pl.ds((base + r) * T, T)], osems[b])

    pltpu.make_async_copy(bufs[(RPW - 2) % 2], o_hbm.at[pl.ds(0, T)],
                          osems[(RPW - 2) % 2]).wait()
    pltpu.make_async_copy(bufs[(RPW - 1) % 2], o_hbm.at[pl.ds(0, T)],
                          osems[(RPW - 1) % 2]).wait()


def kernel(payload, seq_lens, bn_weight, bn_bias):
    lens = seq_lens.astype(jnp.int32)
    lensx = jnp.repeat(lens, LN)
    mesh = plsc.VectorSubcoreMesh(core_axis_name="c", subcore_axis_name="s")

    stats = functools.partial(
        pl.kernel,
        out_type=jax.ShapeDtypeStruct((2 * NW, LN), jnp.float32),
        mesh=mesh,
        scratch_types=[
            pltpu.VMEM((RPW * LN,), jnp.int32),
            pltpu.VMEM((T,), jnp.float32),
            pltpu.VMEM((T,), jnp.float32),
            pltpu.VMEM((2, LN), jnp.float32),
            pltpu.SemaphoreType.DMA,
            pltpu.SemaphoreType.DMA,
        ],
    )(_stats_body)
    partials = stats(payload, lensx)

    total_s = jnp.sum(partials[0::2])
    total_q = jnp.sum(partials[1::2])
    n = jnp.sum(lens).astype(jnp.float32)
    mean = total_s / n
    var = jnp.maximum(total_q / n - mean * mean, 0.0)
    rstd = jax.lax.rsqrt(var + 1e-5)
    a = rstd * bn_weight[0]
    c = bn_bias[0] - mean * a
    acv = jnp.stack([jnp.full((LN,), a, jnp.float32),
                     jnp.full((LN,), c, jnp.float32)])

    norm = functools.partial(
        pl.kernel,
        out_type=jax.ShapeDtypeStruct((B * T,), jnp.float32),
        mesh=mesh,
        scratch_types=[
            pltpu.VMEM((RPW * LN,), jnp.int32),
            pltpu.VMEM((2, LN), jnp.float32),
            pltpu.VMEM((T,), jnp.float32),
            pltpu.VMEM((T,), jnp.float32),
            pltpu.SemaphoreType.DMA,
            pltpu.SemaphoreType.DMA,
            pltpu.SemaphoreType.DMA,
            pltpu.SemaphoreType.DMA,
        ],
    )(_norm_body)
    out = norm(payload, lensx, acv)
    return out.reshape(B, T, 1)


# speedup vs baseline: 1.3263x; 1.3263x over previous
import functools

import jax
import jax.numpy as jnp
from jax import lax
from jax.experimental import pallas as pl
from jax.experimental.pallas import tpu as pltpu
from jax.experimental.pallas import tpu_sc as plsc

B, T = 1024, 4096
NW = 32
RPW = B // NW
LN = 16
NV = T // LN
U = 8


def _iotas():
    base = lax.broadcasted_iota(jnp.int32, (LN,), 0)
    return tuple(base + k * LN for k in range(U))


def _wid():
    return lax.axis_index("s") * 2 + lax.axis_index("c")


def _stats_body(x_hbm, lensx_hbm, part_hbm, lxv, buf0, buf1, pbuf, s0, s1):
    base = _wid() * RPW
    pltpu.sync_copy(lensx_hbm.at[pl.ds(base * LN, RPW * LN)], lxv)
    bufs, sems = (buf0, buf1), (s0, s1)

    sv = jnp.zeros((LN,), jnp.float32)
    qv = jnp.zeros((LN,), jnp.float32)
    pltpu.async_copy(x_hbm.at[base], buf0, s0)
    for r in range(RPW):
        b = r % 2
        pltpu.make_async_copy(x_hbm.at[base + r], bufs[b], sems[b]).wait()
        if r + 1 < RPW:
            pltpu.async_copy(x_hbm.at[base + r + 1], bufs[(r + 1) % 2],
                             sems[(r + 1) % 2])
        buf = bufs[b]
        lenv = lxv[pl.ds(r * LN, LN)]
        iotas = _iotas()

        def inner(j, carry):
            lv2, s0, s1, q0, q1 = carry
            for k in range(U):
                x = buf[pl.ds(j * (U * LN) + k * LN, LN)]
                xm = jnp.where(iotas[k] < lv2, x, 0.0)
                if k % 2 == 0:
                    s0 = s0 + xm
                    q0 = q0 + xm * x
                else:
                    s1 = s1 + xm
                    q1 = q1 + xm * x
            return lv2 - U * LN, s0, s1, q0, q1

        _, sv, s1, qv, q1 = lax.fori_loop(
            0, NV // U, inner, (lenv, sv, jnp.zeros((LN,), jnp.float32),
                                qv, jnp.zeros((LN,), jnp.float32)))
        sv = sv + s1
        qv = qv + q1

    pbuf[0, :] = sv
    pbuf[1, :] = qv
    pltpu.sync_copy(pbuf, part_hbm.at[pl.ds(2 * _wid(), 2)])


def _norm_body(x_hbm, lensx_hbm, ac_hbm, o_hbm,
               lxv, ac, buf0, buf1, si0, si1, so0, so1):
    base = _wid() * RPW
    pltpu.sync_copy(lensx_hbm.at[pl.ds(base * LN, RPW * LN)], lxv)
    pltpu.sync_copy(ac_hbm, ac)
    a_v = ac[0, :]
    c_v = ac[1, :]

    bufs, isems, osems = (buf0, buf1), (si0, si1), (so0, so1)
    pltpu.async_copy(x_hbm.at[base], buf0, si0)
    for r in range(RPW):
        b = r % 2
        pltpu.make_async_copy(x_hbm.at[base + r], bufs[b], isems[b]).wait()
        if r + 1 < RPW:
            nb = (r + 1) % 2
            if r >= 1:
                pltpu.make_async_copy(bufs[nb], o_hbm.at[pl.ds(0, T)],
                                      osems[nb]).wait()
            pltpu.async_copy(x_hbm.at[base + r + 1], bufs[nb], isems[nb])
        buf = bufs[b]
        lenv = lxv[pl.ds(r * LN, LN)]
        iotas = _iotas()

        def inner(j, lv2):
            for k in range(U):
                x = buf[pl.ds(j * (U * LN) + k * LN, LN)]
                y = x * a_v + c_v
                buf[pl.ds(j * (U * LN) + k * LN, LN)] = jnp.where(
                    iotas[k] < lv2, y, x)
            return lv2 - U * LN

        lax.fori_loop(0, NV // U, inner, lenv)
        pltpu.async_copy(buf, o_hbm.at[pl.ds((base + r) * T, T)], osems[b])

    pltpu.make_async_copy(bufs[(RPW - 2) % 2], o_hbm.at[pl.ds(0, T)],
                          osems[(RPW - 2) % 2]).wait()
    pltpu.make_async_copy(bufs[(RPW - 1) % 2], o_hbm.at[pl.ds(0, T)],
                          osems[(RPW - 1) % 2]).wait()


def kernel(payload, seq_lens, bn_weight, bn_bias):
    lens = seq_lens.astype(jnp.int32)
    lensx = jnp.repeat(lens, LN)
    mesh = plsc.VectorSubcoreMesh(core_axis_name="c", subcore_axis_name="s")

    stats = functools.partial(
        pl.kernel,
        out_type=jax.ShapeDtypeStruct((2 * NW, LN), jnp.float32),
        mesh=mesh,
        scratch_types=[
            pltpu.VMEM((RPW * LN,), jnp.int32),
            pltpu.VMEM((T,), jnp.float32),
            pltpu.VMEM((T,), jnp.float32),
            pltpu.VMEM((2, LN), jnp.float32),
            pltpu.SemaphoreType.DMA,
            pltpu.SemaphoreType.DMA,
        ],
    )(_stats_body)
    partials = stats(payload, lensx)

    total_s = jnp.sum(partials[0::2])
    total_q = jnp.sum(partials[1::2])
    n = jnp.sum(lens).astype(jnp.float32)
    mean = total_s / n
    var = jnp.maximum(total_q / n - mean * mean, 0.0)
    rstd = jax.lax.rsqrt(var + 1e-5)
    a = rstd * bn_weight[0]
    c = bn_bias[0] - mean * a
    acv = jnp.stack([jnp.full((LN,), a, jnp.float32),
                     jnp.full((LN,), c, jnp.float32)])

    norm = functools.partial(
        pl.kernel,
        out_type=jax.ShapeDtypeStruct((B * T,), jnp.float32),
        mesh=mesh,
        scratch_types=[
            pltpu.VMEM((RPW * LN,), jnp.int32),
            pltpu.VMEM((2, LN), jnp.float32),
            pltpu.VMEM((T,), jnp.float32),
            pltpu.VMEM((T,), jnp.float32),
            pltpu.SemaphoreType.DMA,
            pltpu.SemaphoreType.DMA,
            pltpu.SemaphoreType.DMA,
            pltpu.SemaphoreType.DMA,
        ],
    )(_norm_body)
    out = norm(payload, lensx, acv)
    return out.reshape(B, T, 1)


# speedup vs baseline: 4.7961x; 3.6162x over previous
import jax
import jax.numpy as jnp
from jax.experimental import pallas as pl
from jax.experimental.pallas import tpu as pltpu

B, T = 1024, 4096
BR = 128
NBLK = B // BR
FR = BR * (T // 128)


def _body(lens_ref, w_ref, b_ref, x_ref, o_ref, stash_ref, acc_ref):
    p = pl.program_id(0)
    i = pl.program_id(1)
    lens = lens_ref[...]

    @pl.when(p == 0)
    def _phase_stats():
        x = x_ref[...]
        col = jax.lax.broadcasted_iota(jnp.int32, x.shape, 1)
        maskf = (col < lens).astype(jnp.float32)
        xm = x * maskf

        @pl.when(i == 0)
        def _init():
            acc_ref[0] = 0.0
            acc_ref[1] = 0.0
            acc_ref[2] = 0.0

        acc_ref[0] += jnp.sum(xm)
        acc_ref[1] += jnp.sum(xm * x)
        acc_ref[2] += jnp.sum(maskf)
        stash_ref[pl.ds(i * BR, BR), :] = x

    @pl.when(p == 1)
    def _phase_norm():
        x = stash_ref[pl.ds(i * BR, BR), :]
        n = acc_ref[2]
        mean = acc_ref[0] / n
        var = jnp.maximum(acc_ref[1] / n - mean * mean, 0.0)
        rstd = jax.lax.rsqrt(var + 1e-5)
        a = rstd * w_ref[0]
        c = b_ref[0] - mean * a
        col = jax.lax.broadcasted_iota(jnp.int32, x.shape, 1)
        y = jnp.where(col < lens, x * a + c, x)
        o_ref[...] = y.reshape(FR, 128)


def kernel(payload, seq_lens, bn_weight, bn_bias):
    lens2 = seq_lens.reshape(B, 1).astype(jnp.int32)
    smem = pl.BlockSpec(memory_space=pltpu.SMEM)
    out = pl.pallas_call(
        _body,
        grid=(2, NBLK),
        in_specs=[
            pl.BlockSpec((BR, 1), lambda p, i: (i, 0)),
            smem,
            smem,
            pl.BlockSpec((BR, T), lambda p, i: (jnp.where(p == 0, i, NBLK - 1), 0)),
        ],
        out_specs=pl.BlockSpec((FR, 128), lambda p, i: (jnp.where(p == 0, 0, i), 0)),
        out_shape=jax.ShapeDtypeStruct((B * (T // 128), 128), jnp.float32),
        scratch_shapes=[
            pltpu.VMEM((B, T), jnp.float32),
            pltpu.SMEM((3,), jnp.float32),
        ],
    )(lens2, bn_weight, bn_bias, payload)
    return out.reshape(B, T, 1)
